# Initial kernel scaffold; baseline (speedup 1.0000x reference)
import functools

import jax
import jax.numpy as jnp
from jax import lax
from jax.experimental import pallas as pl
from jax.experimental.pallas import tpu as pltpu
from jax.experimental.pallas import tpu_sc as plsc

_C = 80
_NC = 2
_NS = 16


def _sc_aggregate(x, src2, dst2, zeros2, zeros1):
    n, d = x.shape
    chunks = src2.shape[0]
    chunks_per_tile = chunks // (_NC * _NS)
    rows_per_tile = n // _NS

    mesh = plsc.VectorSubcoreMesh(core_axis_name="c", subcore_axis_name="s")

    @functools.partial(
        pl.kernel,
        mesh=mesh,
        out_type=[
            jax.ShapeDtypeStruct((_NC, n, d), jnp.float32),
            jax.ShapeDtypeStruct((_NC, n), jnp.float32),
        ],
        scratch_types=[
            pltpu.VMEM((chunks_per_tile, _C), jnp.int32),
            pltpu.VMEM((chunks_per_tile, _C), jnp.int32),
            pltpu.VMEM((_C, d), jnp.float32),
            pltpu.VMEM((_C,), jnp.float32),
            pltpu.VMEM_SHARED((n, d), jnp.float32),
            pltpu.VMEM_SHARED((n,), jnp.float32),
            pltpu.SemaphoreType.DMA,
        ],
    )
    def k(x_hbm, src_hbm, dst_hbm, z2_hbm, z1_hbm, agg_out, deg_out,
          src_v, dst_v, rows_v, ones_v, agg_sh, deg_sh, sem):
        c = lax.axis_index("c")
        s = lax.axis_index("s")

        pltpu.sync_copy(
            z2_hbm.at[pl.ds(s * rows_per_tile, rows_per_tile)],
            agg_sh.at[pl.ds(s * rows_per_tile, rows_per_tile)])

        @pl.when(s == 0)
        def _():
            pltpu.sync_copy(z1_hbm, deg_sh)

        base = (c * _NS + s) * chunks_per_tile
        pltpu.sync_copy(src_hbm.at[pl.ds(base, chunks_per_tile)], src_v)
        pltpu.sync_copy(dst_hbm.at[pl.ds(base, chunks_per_tile)], dst_v)

        for i in range(_C // 16):
            ones_v[pl.ds(i * 16, 16)] = jnp.ones((16,), jnp.float32)

        plsc.subcore_barrier()

        def body(j, carry):
            pltpu.async_copy(x_hbm.at[src_v.at[j]], rows_v, sem).wait()
            pltpu.sync_copy(rows_v, agg_sh.at[dst_v.at[j]], add=True)
            pltpu.sync_copy(ones_v, deg_sh.at[dst_v.at[j]], add=True)
            return carry

        lax.fori_loop(0, chunks_per_tile, body, 0)

        plsc.subcore_barrier()

        pltpu.sync_copy(
            agg_sh.at[pl.ds(s * rows_per_tile, rows_per_tile)],
            agg_out.at[c, pl.ds(s * rows_per_tile, rows_per_tile)])

        @pl.when(s == 0)
        def _():
            pltpu.sync_copy(deg_sh, deg_out.at[c])

    return k(x, src2, dst2, zeros2, zeros1)


def _tc_body(x_ref, agg_ref, deg_ref, ws_ref, wn_ref, b_ref, o_ref):
    agg = agg_ref[0] + agg_ref[1]
    deg = jnp.maximum(deg_ref[0] + deg_ref[1], 1.0)
    h = agg / deg
    o_ref[...] = (
        jnp.dot(x_ref[...], ws_ref[...], preferred_element_type=jnp.float32)
        + jnp.dot(h, wn_ref[...], preferred_element_type=jnp.float32)
        + b_ref[...])


def kernel(x, edge_index, W_self, W_neigh, b):
    n, d = x.shape
    e = edge_index.shape[1]
    f = W_self.shape[1]

    ei = edge_index.astype(jnp.int32)
    src2 = ei[0].reshape(e // _C, _C)
    dst2 = ei[1].reshape(e // _C, _C)
    zeros2 = jnp.zeros((n, d), jnp.float32)
    zeros1 = jnp.zeros((n,), jnp.float32)

    agg2, deg2 = _sc_aggregate(x, src2, dst2, zeros2, zeros1)
    deg2 = deg2.reshape(_NC, n, 1)

    out = pl.pallas_call(
        _tc_body,
        out_shape=jax.ShapeDtypeStruct((n, f), jnp.float32),
    )(x, agg2, deg2, W_self, W_neigh, b.reshape(1, f))
    return out


# speedup vs baseline: 8.2409x; 8.2409x over previous
import functools

import jax
import jax.numpy as jnp
from jax import lax
from jax.experimental import pallas as pl
from jax.experimental.pallas import tpu as pltpu
from jax.experimental.pallas import tpu_sc as plsc

_C = 80
_NC = 2
_NS = 16


def _sc_aggregate(x, src3, dst3, zeros2, zeros1):
    n, d = x.shape
    chunks_per_tile = src3.shape[1]

    mesh = plsc.VectorSubcoreMesh(core_axis_name="c", subcore_axis_name="s")

    @functools.partial(
        pl.kernel,
        mesh=mesh,
        out_type=[
            jax.ShapeDtypeStruct((n, d), jnp.float32),
            jax.ShapeDtypeStruct((n, d), jnp.float32),
            jax.ShapeDtypeStruct((n,), jnp.float32),
            jax.ShapeDtypeStruct((n,), jnp.float32),
        ],
        scratch_types=[
            pltpu.VMEM((chunks_per_tile, _C), jnp.int32),
            pltpu.VMEM((chunks_per_tile, _C), jnp.int32),
            pltpu.VMEM((_C, d), jnp.float32),
            pltpu.VMEM((_C,), jnp.float32),
            pltpu.VMEM_SHARED((n, d), jnp.float32),
            pltpu.VMEM_SHARED((n,), jnp.float32),
            pltpu.SemaphoreType.DMA,
        ],
    )
    def k(x_hbm, src_hbm, dst_hbm, z2_hbm, z1_hbm,
          agg0_out, agg1_out, deg0_out, deg1_out,
          src_v, dst_v, rows_v, ones_v, agg_sh, deg_sh, sem):
        c = lax.axis_index("c")
        s = lax.axis_index("s")
        w = c * _NS + s

        @pl.when(s == 0)
        def _():
            pltpu.sync_copy(z2_hbm, agg_sh)

        @pl.when(s == 1)
        def _():
            pltpu.sync_copy(z1_hbm, deg_sh)

        pltpu.sync_copy(src_hbm.at[w], src_v)
        pltpu.sync_copy(dst_hbm.at[w], dst_v)

        for i in range(_C // 16):
            ones_v[pl.ds(i * 16, 16)] = jnp.ones((16,), jnp.float32)

        plsc.subcore_barrier()

        def body(j, carry):
            pltpu.async_copy(x_hbm.at[src_v.at[j]], rows_v, sem).wait()
            pltpu.sync_copy(rows_v, agg_sh.at[dst_v.at[j]], add=True)
            pltpu.sync_copy(ones_v, deg_sh.at[dst_v.at[j]], add=True)
            return carry

        lax.fori_loop(0, chunks_per_tile, body, 0)

        plsc.subcore_barrier()

        @pl.when(jnp.logical_and(s == 0, c == 0))
        def _():
            pltpu.sync_copy(agg_sh, agg0_out)

        @pl.when(jnp.logical_and(s == 0, c == 1))
        def _():
            pltpu.sync_copy(agg_sh, agg1_out)

        @pl.when(jnp.logical_and(s == 1, c == 0))
        def _():
            pltpu.sync_copy(deg_sh, deg0_out)

        @pl.when(jnp.logical_and(s == 1, c == 1))
        def _():
            pltpu.sync_copy(deg_sh, deg1_out)

    return k(x, src3, dst3, zeros2, zeros1)


def _tc_body(x_ref, agg0_ref, agg1_ref, deg0_ref, deg1_ref,
             ws_ref, wn_ref, b_ref, o_ref):
    agg = agg0_ref[...] + agg1_ref[...]
    deg = jnp.maximum(deg0_ref[...] + deg1_ref[...], 1.0)
    h = agg / deg
    o_ref[...] = (
        jnp.dot(x_ref[...], ws_ref[...], preferred_element_type=jnp.float32)
        + jnp.dot(h, wn_ref[...], preferred_element_type=jnp.float32)
        + b_ref[...])


def kernel(x, edge_index, W_self, W_neigh, b):
    n, d = x.shape
    e = edge_index.shape[1]
    f = W_self.shape[1]
    nw = _NC * _NS

    ei = edge_index.astype(jnp.int32)
    src3 = ei[0].reshape(nw, e // (nw * _C), _C)
    dst3 = ei[1].reshape(nw, e // (nw * _C), _C)
    zeros2 = jnp.zeros((n, d), jnp.float32)
    zeros1 = jnp.zeros((n,), jnp.float32)

    agg0, agg1, deg0, deg1 = _sc_aggregate(x, src3, dst3, zeros2, zeros1)

    out = pl.pallas_call(
        _tc_body,
        out_shape=jax.ShapeDtypeStruct((n, f), jnp.float32),
    )(x, agg0, agg1, deg0.reshape(n, 1), deg1.reshape(n, 1),
      W_self, W_neigh, b.reshape(1, f))
    return out
